# initial kernel scaffold (unmeasured)
import functools

import jax
import jax.numpy as jnp
from jax import lax
from jax.experimental import pallas as pl
from jax.experimental.pallas import tpu as pltpu

N_EXPERTS = 8
E_LOCAL = 4
C = 640
FC = 512


def _a2a_exchange(buf, collective_id):

    def body(b_ref, o_ref, send_sem, recv_sem):
        my_x = lax.axis_index("x")
        my_y = lax.axis_index("y")
        my_z = lax.axis_index("z")
        partner = (1 - my_x, my_y, my_z)

        barrier = pltpu.get_barrier_semaphore()
        pl.semaphore_signal(
            barrier, inc=1, device_id=partner,
            device_id_type=pl.DeviceIdType.MESH,
        )
        pl.semaphore_wait(barrier, 1)

        rdma = pltpu.make_async_remote_copy(
            src_ref=b_ref,
            dst_ref=o_ref,
            send_sem=send_sem,
            recv_sem=recv_sem,
            device_id=partner,
            device_id_type=pl.DeviceIdType.MESH,
        )
        rdma.start()
        rdma.wait()

    return pl.pallas_call(
        body,
        out_shape=jax.ShapeDtypeStruct(buf.shape, buf.dtype),
        in_specs=[pl.BlockSpec(memory_space=pltpu.VMEM)],
        out_specs=pl.BlockSpec(memory_space=pltpu.VMEM),
        scratch_shapes=[pltpu.SemaphoreType.DMA, pltpu.SemaphoreType.DMA],
        compiler_params=pltpu.CompilerParams(collective_id=collective_id),
    )(buf)


def _expert_ffn(xcat, W1, W2):
    n_e, rows, d = xcat.shape
    f = W1.shape[2]
    nk = f // FC

    def body(x_ref, w1_ref, w2_ref, o_ref):
        k = pl.program_id(1)
        h = jnp.maximum(
            jnp.dot(x_ref[0], w1_ref[0], preferred_element_type=jnp.float32),
            0.0,
        )
        contrib = jnp.dot(h, w2_ref[0], preferred_element_type=jnp.float32)

        @pl.when(k == 0)
        def _():
            o_ref[0] = contrib

        @pl.when(k != 0)
        def _():
            o_ref[0] += contrib

    return pl.pallas_call(
        body,
        grid=(n_e, nk),
        in_specs=[
            pl.BlockSpec((1, rows, d), lambda e, k: (e, 0, 0)),
            pl.BlockSpec((1, d, FC), lambda e, k: (e, 0, k)),
            pl.BlockSpec((1, FC, d), lambda e, k: (e, k, 0)),
        ],
        out_specs=pl.BlockSpec((1, rows, d), lambda e, k: (e, 0, 0)),
        out_shape=jax.ShapeDtypeStruct((n_e, rows, d), jnp.float32),
    )(xcat, W1, W2)


def kernel(x, assign, W1, W2):
    T, D = x.shape
    my_x = lax.axis_index("x")
    my_base = my_x * E_LOCAL
    partner_base = (1 - my_x) * E_LOCAL

    idx_all = jnp.stack(
        [jnp.nonzero(assign == e, size=C, fill_value=T)[0]
         for e in range(N_EXPERTS)]
    )
    xp = jnp.concatenate([x, jnp.zeros((1, D), x.dtype)], axis=0)
    bufs = xp[idx_all]

    send_buf = lax.dynamic_slice_in_dim(bufs, partner_base, E_LOCAL, axis=0)
    local_buf = lax.dynamic_slice_in_dim(bufs, my_base, E_LOCAL, axis=0)

    recv_buf = _a2a_exchange(send_buf, collective_id=0)

    xcat = jnp.concatenate([local_buf, recv_buf], axis=1)
    y = _expert_ffn(xcat, W1, W2)
    y_local = y[:, :C]
    y_remote = y[:, C:]

    y_back = _a2a_exchange(y_remote, collective_id=1)

    idx_mine = lax.dynamic_slice_in_dim(idx_all, my_base, E_LOCAL, axis=0)
    idx_sent = lax.dynamic_slice_in_dim(idx_all, partner_base, E_LOCAL, axis=0)
    outp = jnp.zeros((T + 1, D), jnp.float32)
    outp = outp.at[idx_mine.reshape(-1)].add(y_local.reshape(-1, D))
    outp = outp.at[idx_sent.reshape(-1)].add(y_back.reshape(-1, D))
    return outp[:T]


# baseline (device time: 2736580 ns/iter reference)
import functools

import jax
import jax.numpy as jnp
from jax import lax
from jax.experimental import pallas as pl
from jax.experimental.pallas import tpu as pltpu

N_EXPERTS = 8
E_LOCAL = 4
C = 640
FC = 512


def _a2a_exchange(buf, collective_id):

    def body(b_ref, o_ref, send_sem, recv_sem):
        my_x = lax.axis_index("x")
        my_y = lax.axis_index("y")
        my_z = lax.axis_index("z")
        partner = (1 - my_x, my_y, my_z)

        barrier = pltpu.get_barrier_semaphore()
        pl.semaphore_signal(
            barrier, inc=1, device_id=partner,
            device_id_type=pl.DeviceIdType.MESH,
        )
        pl.semaphore_wait(barrier, 1)

        rdma = pltpu.make_async_remote_copy(
            src_ref=b_ref,
            dst_ref=o_ref,
            send_sem=send_sem,
            recv_sem=recv_sem,
            device_id=partner,
            device_id_type=pl.DeviceIdType.MESH,
        )
        rdma.start()
        rdma.wait()

    return pl.pallas_call(
        body,
        out_shape=jax.ShapeDtypeStruct(buf.shape, buf.dtype),
        in_specs=[pl.BlockSpec(memory_space=pltpu.VMEM)],
        out_specs=pl.BlockSpec(memory_space=pltpu.VMEM),
        scratch_shapes=[pltpu.SemaphoreType.DMA, pltpu.SemaphoreType.DMA],
        compiler_params=pltpu.CompilerParams(collective_id=collective_id),
    )(buf)


def _expert_ffn(xcat, W1, W2):
    n_e, rows, d = xcat.shape
    f = W1.shape[2]
    nk = f // FC
    R = 640
    nr = rows // R

    def body(x_ref, w1_ref, w2_ref, o_ref):
        k = pl.program_id(2)
        h = jnp.maximum(
            jnp.dot(x_ref[0], w1_ref[0], preferred_element_type=jnp.float32),
            0.0,
        )
        contrib = jnp.dot(h, w2_ref[0], preferred_element_type=jnp.float32)

        @pl.when(k == 0)
        def _():
            o_ref[0] = contrib

        @pl.when(k != 0)
        def _():
            o_ref[0] += contrib

    return pl.pallas_call(
        body,
        grid=(n_e, nr, nk),
        in_specs=[
            pl.BlockSpec((1, R, d), lambda e, r, k: (e, r, 0)),
            pl.BlockSpec((1, d, FC), lambda e, r, k: (e, 0, k)),
            pl.BlockSpec((1, FC, d), lambda e, r, k: (e, k, 0)),
        ],
        out_specs=pl.BlockSpec((1, R, d), lambda e, r, k: (e, r, 0)),
        out_shape=jax.ShapeDtypeStruct((n_e, rows, d), jnp.float32),
        compiler_params=pltpu.CompilerParams(
            vmem_limit_bytes=56 * 1024 * 1024
        ),
    )(xcat, W1, W2)


def kernel(x, assign, W1, W2):
    T, D = x.shape
    my_x = lax.axis_index("x")
    my_base = my_x * E_LOCAL
    partner_base = (1 - my_x) * E_LOCAL

    idx_all = jnp.stack(
        [jnp.nonzero(assign == e, size=C, fill_value=T)[0]
         for e in range(N_EXPERTS)]
    )
    xp = jnp.concatenate([x, jnp.zeros((1, D), x.dtype)], axis=0)
    bufs = xp[idx_all]

    send_buf = lax.dynamic_slice_in_dim(bufs, partner_base, E_LOCAL, axis=0)
    local_buf = lax.dynamic_slice_in_dim(bufs, my_base, E_LOCAL, axis=0)

    recv_buf = _a2a_exchange(send_buf, collective_id=0)

    xcat = jnp.concatenate([local_buf, recv_buf], axis=1)
    y = _expert_ffn(xcat, W1, W2)
    y_local = y[:, :C]
    y_remote = y[:, C:]

    y_back = _a2a_exchange(y_remote, collective_id=1)

    idx_mine = lax.dynamic_slice_in_dim(idx_all, my_base, E_LOCAL, axis=0)
    idx_sent = lax.dynamic_slice_in_dim(idx_all, partner_base, E_LOCAL, axis=0)
    outp = jnp.zeros((T + 1, D), jnp.float32)
    outp = outp.at[idx_mine.reshape(-1)].add(y_local.reshape(-1, D))
    outp = outp.at[idx_sent.reshape(-1)].add(y_back.reshape(-1, D))
    return outp[:T]
